# baseline (speedup 1.0000x reference)
import functools

import jax
import jax.numpy as jnp
from jax import lax
from jax.experimental import pallas as pl
from jax.experimental.pallas import tpu as pltpu
from jax.experimental.pallas import tpu_sc as plsc

_NC = 2
_NS = 16
_NW = _NC * _NS
_L = 16


@functools.lru_cache(maxsize=None)
def _build_gather(B, N, V2, D):
    nb = B // _NW
    n_pairs = N // 2
    n_groups = nb // _L
    mesh = plsc.VectorSubcoreMesh(core_axis_name="c", subcore_axis_name="s")

    @functools.partial(
        pl.kernel,
        mesh=mesh,
        out_type=jax.ShapeDtypeStruct((N, D, B), jnp.float32),
        scratch_types=[
            pltpu.VMEM((N, nb), jnp.int32),
            pltpu.VMEM((N, nb), jnp.int32),
            pltpu.VMEM((nb, 128), jnp.float32),
            pltpu.VMEM((nb, 128), jnp.float32),
            pltpu.VMEM((D, nb), jnp.float32),
            pltpu.VMEM((D, nb), jnp.float32),
            pltpu.VMEM((N // 2, 128), jnp.float32),
            pltpu.SemaphoreType.DMA,
            pltpu.SemaphoreType.DMA,
            pltpu.SemaphoreType.DMA,
            pltpu.SemaphoreType.DMA,
        ],
        compiler_params=pltpu.CompilerParams(use_tc_tiling_on_sc=True,
                                             needs_layout_passes=False),
    )
    def g(idx_hbm, tok_hbm, pos_hbm, out_hbm,
          idx_v, pid_v, r0, r1, o0, o1, pos_v, g0, g1, s0, s1):
        wid = lax.axis_index("s") * _NC + lax.axis_index("c")
        b0 = wid * nb
        rows = (r0, r1)
        obuf = (o0, o1)
        gsem = (g0, g1)
        osem = (s0, s1)

        pltpu.sync_copy(idx_hbm.at[:, pl.ds(b0, nb)], idx_v)
        pltpu.sync_copy(pos_hbm, pos_v)

        def mk_pid(n, carry):
            for grp in range(n_groups):
                sl = pl.ds(grp * _L, _L)
                pid_v[n, sl] = lax.shift_right_logical(idx_v[n, sl], 1)
            return carry

        lax.fori_loop(0, N, mk_pid, 0)

        def issue_gather(n, b):
            pltpu.async_copy(tok_hbm.at[pid_v.at[n]], rows[b], gsem[b])

        def wait_gather(n, b):
            pltpu.make_async_copy(tok_hbm.at[pid_v.at[n]], rows[b],
                                  gsem[b]).wait()

        def issue_writeout(n, b):
            pltpu.async_copy(obuf[b], out_hbm.at[n, :, pl.ds(b0, nb)],
                             osem[b])

        def wait_writeout(b):
            pltpu.make_async_copy(obuf[b], out_hbm.at[0, :, pl.ds(b0, nb)],
                                  osem[b]).wait()

        iota = lax.broadcasted_iota(jnp.int32, (_L,), 0)

        def transpose_add(j, n, b):
            def add_pos(i, carry):
                for c in range(D // _L):
                    psl = pl.ds(b * D + c * _L, _L)
                    pv = pos_v[j, psl]
                    lo = pl.ds(c * _L, _L)
                    hi = pl.ds(D + c * _L, _L)
                    rows[b][i, lo] = rows[b][i, lo] + pv
                    rows[b][i, hi] = rows[b][i, hi] + pv
                return carry

            lax.fori_loop(0, nb, add_pos, 0)

            for grp in range(n_groups):
                sl = pl.ds(grp * _L, _L)
                row_ids = iota + (grp * _L)
                par64 = (idx_v[n, sl] & 1) * 64

                def per_d(d, carry):
                    col_ids = par64 + d
                    val = plsc.load_gather(rows[b], [row_ids, col_ids])
                    obuf[b][d, sl] = val
                    return carry

                lax.fori_loop(0, D, per_d, 0)

        issue_gather(0, 0)
        issue_gather(1, 1)

        def pair_body(j, carry):
            for b in range(2):
                n = 2 * j + b
                wait_gather(n, b)

                @pl.when(n >= 2)
                def _():
                    wait_writeout(b)

                transpose_add(j, n, b)
                issue_writeout(n, b)

                @pl.when(n + 2 < N)
                def _():
                    issue_gather(n + 2, b)

            return carry

        lax.fori_loop(0, n_pairs, pair_body, 0)
        wait_writeout(0)
        wait_writeout(1)

    return g


def kernel(indices, tok_emb_table, pos_emb_table):
    B, N = indices.shape
    V, D = tok_emb_table.shape
    idx_t = indices.T.astype(jnp.int32)
    tokr = tok_emb_table.reshape(V // 2, 2 * D)
    pos2 = pos_emb_table[:N].astype(jnp.float32).reshape(N // 2, 2 * D)
    outp = _build_gather(B, N, V // 2, D)(idx_t, tokr, pos2)
    return jnp.transpose(outp, (2, 0, 1))
